# baseline (speedup 1.0000x reference)
import functools

import jax
import jax.numpy as jnp
from jax import lax
from jax.experimental import pallas as pl
from jax.experimental.pallas import tpu as pltpu
from jax.experimental.pallas import tpu_sc as plsc

L = 16
C = 4
S = 2
PASS_BITS = (11, 11, 10)
PASS_SHIFTS = (0, 11, 22)
NBINS_MAX = 1 << max(PASS_BITS)
COMB_MAX = NBINS_MAX * C
SIGN = -0x80000000


def _i32(x):
    return plsc.bitcast(x, jnp.int32)


def _f32(x):
    return plsc.bitcast(x, jnp.float32)


@functools.partial(jax.jit, static_argnums=(2, 3))
def _sc_sort(x_flat, rev_vec, rows, n):
    info = plsc.get_sparse_core_info()
    nc, ns = info.num_cores, info.num_subcores
    nw = nc * ns
    rpw = rows // nw
    seg = n // L
    tb = seg // C
    rsh = (n // C).bit_length() - 1
    csh = C.bit_length() - 1

    mesh = plsc.VectorSubcoreMesh(core_axis_name="c", subcore_axis_name="s")

    @functools.partial(
        pl.kernel,
        out_type=(
            jax.ShapeDtypeStruct((rows * n,), jnp.float32),
            jax.ShapeDtypeStruct((rows * n,), jnp.int32),
        ),
        mesh=mesh,
        compiler_params=pltpu.CompilerParams(needs_layout_passes=False),
        scratch_types=[
            pltpu.VMEM((n,), jnp.float32),
            pltpu.VMEM((n,), jnp.int32),
            pltpu.VMEM((n,), jnp.float32),
            pltpu.VMEM((NBINS_MAX,), jnp.int32),
            pltpu.VMEM((NBINS_MAX,), jnp.int32),
            pltpu.VMEM((NBINS_MAX,), jnp.int32),
            pltpu.VMEM((NBINS_MAX,), jnp.int32),
            pltpu.VMEM((COMB_MAX,), jnp.int32),
            pltpu.VMEM((COMB_MAX,), jnp.int32),
            pltpu.VMEM((L,), jnp.int32),
        ],
    )
    def sortk(x_hbm, rev_hbm, vals_hbm, idx_hbm,
              keyu, buf_a, buf_b, h0, h1, h2, h3, g0, g1, revv):
        hists = (h0, h1, h2, h3)
        accs = (g0, g1)
        wid = lax.axis_index("s") * nc + lax.axis_index("c")
        pltpu.sync_copy(rev_hbm, revv)
        xm = jnp.where(revv[...] != 0, jnp.full((L,), -1, jnp.int32),
                       jnp.zeros((L,), jnp.int32))
        lane = lax.iota(jnp.int32, L)
        lane_mod = lane & (C - 1)
        lane_div = lane >> csh

        @plsc.parallel_loop(0, COMB_MAX // L, unroll=4)
        def gz_body(i):
            z = jnp.zeros((L,), jnp.int32)
            for g in accs:
                g[pl.ds(i * L, L)] = z

        def load_iv_key(c, t, src, src_is_f32):
            base = (c * tb + t) * L
            if src is None:
                iv = base + lane
                key = _i32(keyu[pl.ds(base, L)])
            else:
                iv = src[pl.ds(base, L)]
                if src_is_f32:
                    iv = _i32(iv)
                key = _i32(plsc.load_gather(keyu, [iv]))
            return iv, key

        def scan0_body(i, carry):
            hv = [h[pl.ds(i * L, L)] for h in hists]
            tot = hv[0] + hv[1] + hv[2] + hv[3]
            cum = plsc.cumsum(tot)
            b = cum - tot + carry
            for c, h in enumerate(hists):
                h[pl.ds(i * L, L)] = b
                if c < C - 1:
                    b = b + hv[c]
            return carry + cum[L - 1]

        def next_scan(nb_next):
            def body(j, carry):
                v0 = g0[pl.ds(j * L, L)]
                v1 = g1[pl.ds(j * L, L)]
                tot = v0 + v1
                cum = plsc.cumsum(tot)
                b = cum - tot + carry
                z = jnp.zeros((L,), jnp.int32)
                g0[pl.ds(j * L, L)] = z
                g1[pl.ds(j * L, L)] = z
                didx = j * (L // C) + lane_div
                for cn, h in enumerate(hists):
                    plsc.store_scatter(h, [didx], b, mask=lane_mod == cn)
                return carry + cum[L - 1]

            lax.fori_loop(0, nb_next * C // L, body, jnp.int32(0), unroll=2)

        def perm_pass(pidx, src, src_is_f32, dst, dst_is_f32, fuse_next):
            shift = PASS_SHIFTS[pidx]
            dmask = (1 << PASS_BITS[pidx]) - 1
            if fuse_next:
                nshift = PASS_SHIFTS[pidx + 1]
                ndmask = (1 << PASS_BITS[pidx + 1]) - 1

            def body(t, _):
                for c, h in enumerate(hists):
                    iv, key = load_iv_key(c, t, src, src_is_f32)
                    d = (key >> shift) & dmask
                    cnt, last = plsc.scan_count(d)
                    base = plsc.load_gather(h, [d])
                    rank = base + cnt - 1
                    plsc.store_scatter(dst, [rank],
                                       _f32(iv) if dst_is_f32 else iv)
                    plsc.addupdate_scatter(h, [d], cnt, mask=last)
                    if fuse_next:
                        nd = (key >> nshift) & ndmask
                        comb = (nd << csh) | (rank >> rsh)
                        cnt2, last2 = plsc.scan_count(comb)
                        plsc.addupdate_scatter(accs[c & (S - 1)], [comb],
                                               cnt2, mask=last2)
                return 0

            lax.fori_loop(0, tb, body, 0, unroll=2)

        def row_body(r, _):
            row = wid * rpw + r
            hbase = row * n
            pltpu.sync_copy(x_hbm.at[pl.ds(hbase, n)], keyu)

            @plsc.parallel_loop(0, NBINS_MAX // L, unroll=4)
            def hz_body(i):
                z = jnp.zeros((L,), jnp.int32)
                for h in hists:
                    h[pl.ds(i * L, L)] = z

            @plsc.parallel_loop(0, tb, unroll=2)
            def tf_body(t):
                for c, h in enumerate(hists):
                    off = (c * tb + t) * L
                    b = _i32(keyu[pl.ds(off, L)])
                    u = (b ^ ((b >> 31) | SIGN)) ^ xm
                    keyu[pl.ds(off, L)] = _f32(u)
                    d = u & ((1 << PASS_BITS[0]) - 1)
                    cnt, last = plsc.scan_count(d)
                    plsc.addupdate_scatter(h, [d], cnt, mask=last)

            lax.fori_loop(0, (1 << PASS_BITS[0]) // L, scan0_body,
                          jnp.int32(0), unroll=2)

            perm_pass(0, None, False, buf_a, False, True)
            next_scan(1 << PASS_BITS[1])
            perm_pass(1, buf_a, False, buf_b, True, True)
            next_scan(1 << PASS_BITS[2])
            perm_pass(2, buf_b, True, buf_a, False, False)

            pltpu.sync_copy(buf_a, idx_hbm.at[pl.ds(hbase, n)])

            @plsc.parallel_loop(0, seg, unroll=4)
            def val_body(i):
                iv = buf_a[pl.ds(i * L, L)]
                v = _i32(plsc.load_gather(keyu, [iv])) ^ xm
                b = v ^ (~(v >> 31) | SIGN)
                buf_b[pl.ds(i * L, L)] = _f32(b)

            pltpu.sync_copy(buf_b, vals_hbm.at[pl.ds(hbase, n)])
            return 0

        lax.fori_loop(0, rpw, row_body, 0)

    return sortk(x_flat, rev_vec)


def kernel(x, reverse):
    rows, n = x.shape
    rev_vec = jnp.full((L,), reverse, dtype=jnp.int32)
    vals, idx = _sc_sort(x.reshape(-1), rev_vec, rows, n)
    return vals.reshape(rows, n), idx.reshape(rows, n)


# speedup vs baseline: 1.2044x; 1.2044x over previous
import functools

import jax
import jax.numpy as jnp
from jax import lax
from jax.experimental import pallas as pl
from jax.experimental.pallas import tpu as pltpu
from jax.experimental.pallas import tpu_sc as plsc

L = 16
C = 4
PASS_BITS = (11, 11, 10)
PASS_SHIFTS = (0, 11, 22)
NBINS_MAX = 1 << max(PASS_BITS)
SIGN = -0x80000000


def _i32(x):
    return plsc.bitcast(x, jnp.int32)


def _f32(x):
    return plsc.bitcast(x, jnp.float32)


@functools.partial(jax.jit, static_argnums=(2, 3))
def _sc_sort(x_flat, rev_vec, rows, n):
    info = plsc.get_sparse_core_info()
    nc, ns = info.num_cores, info.num_subcores
    nw = nc * ns
    rpw = rows // nw
    seg = n // L
    tb = seg // C

    mesh = plsc.VectorSubcoreMesh(core_axis_name="c", subcore_axis_name="s")

    @functools.partial(
        pl.kernel,
        out_type=(
            jax.ShapeDtypeStruct((rows * n,), jnp.float32),
            jax.ShapeDtypeStruct((rows * n,), jnp.int32),
        ),
        mesh=mesh,
        compiler_params=pltpu.CompilerParams(needs_layout_passes=False),
        scratch_types=[
            pltpu.VMEM((n,), jnp.float32),
            pltpu.VMEM((n,), jnp.int32),
            pltpu.VMEM((n,), jnp.float32),
            pltpu.VMEM((NBINS_MAX,), jnp.int32),
            pltpu.VMEM((NBINS_MAX,), jnp.int32),
            pltpu.VMEM((NBINS_MAX,), jnp.int32),
            pltpu.VMEM((NBINS_MAX,), jnp.int32),
            pltpu.VMEM((L,), jnp.int32),
        ],
    )
    def sortk(x_hbm, rev_hbm, vals_hbm, idx_hbm,
              keyu, buf_a, buf_b, h0, h1, h2, h3, revv):
        hists = (h0, h1, h2, h3)
        wid = lax.axis_index("s") * nc + lax.axis_index("c")
        pltpu.sync_copy(rev_hbm, revv)
        xm = jnp.where(revv[...] != 0, jnp.full((L,), -1, jnp.int32),
                       jnp.zeros((L,), jnp.int32))
        lane = lax.iota(jnp.int32, L)

        def run_pass(pidx, src, src_is_f32, dst, dst_is_f32, skip_hist=False):
            shift = PASS_SHIFTS[pidx]
            nb = 1 << PASS_BITS[pidx]
            dmask = nb - 1

            if not skip_hist:
                @plsc.parallel_loop(0, nb // L, unroll=4)
                def zero_body(i):
                    z = jnp.zeros((L,), jnp.int32)
                    for h in hists:
                        h[pl.ds(i * L, L)] = z

            def load_iv_key(c, t):
                base = (c * tb + t) * L
                if src is None:
                    iv = base + lane
                    key = _i32(keyu[pl.ds(base, L)])
                else:
                    iv = src[pl.ds(base, L)]
                    if src_is_f32:
                        iv = _i32(iv)
                    key = _i32(plsc.load_gather(keyu, [iv]))
                return iv, key

            if not skip_hist:
                @plsc.parallel_loop(0, tb, unroll=2)
                def hist_body(t):
                    for c, h in enumerate(hists):
                        _, key = load_iv_key(c, t)
                        d = (key >> shift) & dmask
                        cnt, last = plsc.scan_count(d)
                        plsc.addupdate_scatter(h, [d], cnt, mask=last)

            def scan_body(i, carry):
                hv = [h[pl.ds(i * L, L)] for h in hists]
                tot = hv[0] + hv[1] + hv[2] + hv[3]
                cum = plsc.cumsum(tot)
                b = cum - tot + carry
                for c, h in enumerate(hists):
                    h[pl.ds(i * L, L)] = b
                    if c < C - 1:
                        b = b + hv[c]
                return carry + cum[L - 1]

            lax.fori_loop(0, nb // L, scan_body, jnp.int32(0), unroll=2)

            def perm_body(t, _):
                for c, h in enumerate(hists):
                    iv, key = load_iv_key(c, t)
                    d = (key >> shift) & dmask
                    cnt, last = plsc.scan_count(d)
                    base = plsc.load_gather(h, [d])
                    rank = base + cnt - 1
                    plsc.store_scatter(dst, [rank],
                                       _f32(iv) if dst_is_f32 else iv)
                    plsc.addupdate_scatter(h, [d], cnt, mask=last)
                return 0

            lax.fori_loop(0, tb, perm_body, 0, unroll=2)

        def row_body(r, _):
            row = wid * rpw + r
            hbase = row * n
            pltpu.sync_copy(x_hbm.at[pl.ds(hbase, n)], keyu)

            @plsc.parallel_loop(0, NBINS_MAX // L, unroll=4)
            def hz_body(i):
                z = jnp.zeros((L,), jnp.int32)
                for h in hists:
                    h[pl.ds(i * L, L)] = z

            @plsc.parallel_loop(0, tb, unroll=2)
            def tf_body(t):
                for c, h in enumerate(hists):
                    off = (c * tb + t) * L
                    b = _i32(keyu[pl.ds(off, L)])
                    u = (b ^ ((b >> 31) | SIGN)) ^ xm
                    keyu[pl.ds(off, L)] = _f32(u)
                    d = u & ((1 << PASS_BITS[0]) - 1)
                    cnt, last = plsc.scan_count(d)
                    plsc.addupdate_scatter(h, [d], cnt, mask=last)

            run_pass(0, None, False, buf_a, False, skip_hist=True)
            run_pass(1, buf_a, False, buf_b, True)
            run_pass(2, buf_b, True, buf_a, False)

            pltpu.sync_copy(buf_a, idx_hbm.at[pl.ds(hbase, n)])

            @plsc.parallel_loop(0, seg, unroll=4)
            def val_body(i):
                iv = buf_a[pl.ds(i * L, L)]
                v = _i32(plsc.load_gather(keyu, [iv])) ^ xm
                b = v ^ (~(v >> 31) | SIGN)
                buf_b[pl.ds(i * L, L)] = _f32(b)

            pltpu.sync_copy(buf_b, vals_hbm.at[pl.ds(hbase, n)])
            return 0

        lax.fori_loop(0, rpw, row_body, 0)

    return sortk(x_flat, rev_vec)


def kernel(x, reverse):
    rows, n = x.shape
    rev_vec = jnp.full((L,), reverse, dtype=jnp.int32)
    vals, idx = _sc_sort(x.reshape(-1), rev_vec, rows, n)
    return vals.reshape(rows, n), idx.reshape(rows, n)


# speedup vs baseline: 1.4092x; 1.1701x over previous
import functools

import jax
import jax.numpy as jnp
from jax import lax
from jax.experimental import pallas as pl
from jax.experimental.pallas import tpu as pltpu
from jax.experimental.pallas import tpu_sc as plsc

L = 16
C = 4
PASS_BITS = (11, 11, 10)
PASS_SHIFTS = (0, 11, 22)
NBINS_MAX = 1 << max(PASS_BITS)
SIGN = -0x80000000


def _i32(x):
    return plsc.bitcast(x, jnp.int32)


def _f32(x):
    return plsc.bitcast(x, jnp.float32)


@functools.partial(jax.jit, static_argnums=(2, 3))
def _sc_sort(x_flat, rev_vec, rows, n):
    info = plsc.get_sparse_core_info()
    nc, ns = info.num_cores, info.num_subcores
    nw = nc * ns
    rpw = rows // nw
    seg = n // L
    tb = seg // C
    IVBITS = (n - 1).bit_length()
    IVMASK = (1 << IVBITS) - 1
    assert IVBITS + max(PASS_BITS) <= 31

    mesh = plsc.VectorSubcoreMesh(core_axis_name="c", subcore_axis_name="s")

    @functools.partial(
        pl.kernel,
        out_type=(
            jax.ShapeDtypeStruct((rows * n,), jnp.float32),
            jax.ShapeDtypeStruct((rows * n,), jnp.int32),
        ),
        mesh=mesh,
        compiler_params=pltpu.CompilerParams(needs_layout_passes=False),
        scratch_types=[
            pltpu.VMEM((n,), jnp.float32),
            pltpu.VMEM((n,), jnp.int32),
            pltpu.VMEM((n,), jnp.float32),
            pltpu.VMEM((NBINS_MAX,), jnp.int32),
            pltpu.VMEM((NBINS_MAX,), jnp.int32),
            pltpu.VMEM((NBINS_MAX,), jnp.int32),
            pltpu.VMEM((NBINS_MAX,), jnp.int32),
            pltpu.VMEM((L,), jnp.int32),
        ],
    )
    def sortk(x_hbm, rev_hbm, vals_hbm, idx_hbm,
              keyu, buf_a, buf_b, h0, h1, h2, h3, revv):
        hists = (h0, h1, h2, h3)
        wid = lax.axis_index("s") * nc + lax.axis_index("c")
        pltpu.sync_copy(rev_hbm, revv)
        xm = jnp.where(revv[...] != 0, jnp.full((L,), -1, jnp.int32),
                       jnp.zeros((L,), jnp.int32))
        lane = lax.iota(jnp.int32, L)

        def run_pass(pidx, src, src_is_f32, dst, dst_is_f32,
                     skip_hist=False, pack_next=False):
            shift = PASS_SHIFTS[pidx]
            nb = 1 << PASS_BITS[pidx]
            dmask = nb - 1
            if pack_next:
                nshift = PASS_SHIFTS[pidx + 1]
                ndmask = (1 << PASS_BITS[pidx + 1]) - 1

            if not skip_hist:
                @plsc.parallel_loop(0, nb // L, unroll=4)
                def zero_body(i):
                    z = jnp.zeros((L,), jnp.int32)
                    for h in hists:
                        h[pl.ds(i * L, L)] = z

            def load_d(c, t):
                base = (c * tb + t) * L
                if src is None:
                    return (_i32(keyu[pl.ds(base, L)]) >> shift) & dmask
                pk = src[pl.ds(base, L)]
                if src_is_f32:
                    pk = _i32(pk)
                return pk >> IVBITS

            def load_d_iv_nd(c, t):
                base = (c * tb + t) * L
                if src is None:
                    key = _i32(keyu[pl.ds(base, L)])
                    d = (key >> shift) & dmask
                    iv = base + lane
                    nd = (key >> nshift) & ndmask if pack_next else None
                    return d, iv, nd
                pk = src[pl.ds(base, L)]
                if src_is_f32:
                    pk = _i32(pk)
                d = pk >> IVBITS
                iv = pk & IVMASK
                if pack_next:
                    key = _i32(plsc.load_gather(keyu, [iv]))
                    nd = (key >> nshift) & ndmask
                else:
                    nd = None
                return d, iv, nd

            if not skip_hist:
                @plsc.parallel_loop(0, tb, unroll=2)
                def hist_body(t):
                    for c, h in enumerate(hists):
                        d = load_d(c, t)
                        cnt, last = plsc.scan_count(d)
                        plsc.addupdate_scatter(h, [d], cnt, mask=last)

            def scan_body(i, carry):
                hv = [h[pl.ds(i * L, L)] for h in hists]
                tot = hv[0] + hv[1] + hv[2] + hv[3]
                cum = plsc.cumsum(tot)
                b = cum - tot + carry
                for c, h in enumerate(hists):
                    h[pl.ds(i * L, L)] = b
                    if c < C - 1:
                        b = b + hv[c]
                return carry + cum[L - 1]

            lax.fori_loop(0, nb // L, scan_body, jnp.int32(0), unroll=2)

            def perm_body(t, _):
                for c, h in enumerate(hists):
                    d, iv, nd = load_d_iv_nd(c, t)
                    cnt, last = plsc.scan_count(d)
                    base = plsc.load_gather(h, [d])
                    rank = base + cnt - 1
                    out = (nd << IVBITS) | iv if pack_next else iv
                    plsc.store_scatter(dst, [rank],
                                       _f32(out) if dst_is_f32 else out)
                    plsc.addupdate_scatter(h, [d], cnt, mask=last)
                return 0

            lax.fori_loop(0, tb, perm_body, 0, unroll=2)

        def row_body(r, _):
            row = wid * rpw + r
            hbase = row * n
            pltpu.sync_copy(x_hbm.at[pl.ds(hbase, n)], keyu)

            @plsc.parallel_loop(0, NBINS_MAX // L, unroll=4)
            def hz_body(i):
                z = jnp.zeros((L,), jnp.int32)
                for h in hists:
                    h[pl.ds(i * L, L)] = z

            @plsc.parallel_loop(0, tb, unroll=2)
            def tf_body(t):
                for c, h in enumerate(hists):
                    off = (c * tb + t) * L
                    b = _i32(keyu[pl.ds(off, L)])
                    u = (b ^ ((b >> 31) | SIGN)) ^ xm
                    keyu[pl.ds(off, L)] = _f32(u)
                    d = u & ((1 << PASS_BITS[0]) - 1)
                    cnt, last = plsc.scan_count(d)
                    plsc.addupdate_scatter(h, [d], cnt, mask=last)

            run_pass(0, None, False, buf_a, False,
                     skip_hist=True, pack_next=True)
            run_pass(1, buf_a, False, buf_b, True, pack_next=True)
            run_pass(2, buf_b, True, buf_a, False)

            pltpu.sync_copy(buf_a, idx_hbm.at[pl.ds(hbase, n)])

            @plsc.parallel_loop(0, seg, unroll=4)
            def val_body(i):
                iv = buf_a[pl.ds(i * L, L)]
                v = _i32(plsc.load_gather(keyu, [iv])) ^ xm
                b = v ^ (~(v >> 31) | SIGN)
                buf_b[pl.ds(i * L, L)] = _f32(b)

            pltpu.sync_copy(buf_b, vals_hbm.at[pl.ds(hbase, n)])
            return 0

        lax.fori_loop(0, rpw, row_body, 0)

    return sortk(x_flat, rev_vec)


def kernel(x, reverse):
    rows, n = x.shape
    rev_vec = jnp.full((L,), reverse, dtype=jnp.int32)
    vals, idx = _sc_sort(x.reshape(-1), rev_vec, rows, n)
    return vals.reshape(rows, n), idx.reshape(rows, n)
